# Initial kernel scaffold; baseline (speedup 1.0000x reference)
import jax
import jax.numpy as jnp
from jax.experimental import pallas as pl


def kernel(x, table):
    raise NotImplementedError("write your pallas kernel here")


# speedup vs baseline: 2.8666x; 2.8666x over previous
import functools

import jax
import jax.numpy as jnp
import numpy as np
from jax import lax
from jax.experimental import pallas as pl
from jax.experimental.pallas import tpu as pltpu
from jax.experimental.pallas import tpu_sc as plsc

MAX_SEQ_LEN = 200
D_MODEL = 128
BATCH = 1024
SEQ_LEN = 200

NUM_CORES = 2
NUM_SUBCORES = 16
NUM_WORKERS = NUM_CORES * NUM_SUBCORES

TOKENS = BATCH * SEQ_LEN
CHUNK = 128
TOKENS_PER_WORKER = TOKENS // NUM_WORKERS
CHUNKS = TOKENS_PER_WORKER // CHUNK
LANES = 16
DVECS = D_MODEL // LANES

SCALE = float(np.float32(np.sqrt(np.float32(D_MODEL))))


def _positional_encoding(seq_length, d_model):
    half = d_model // 2
    positions = np.arange(seq_length)[:, np.newaxis]
    d_models = np.arange(half)[np.newaxis, :] / half
    angle_rates = 1.0 / (10000.0 ** d_models)
    angle_rads = positions * angle_rates
    pe = np.concatenate([np.sin(angle_rads), np.cos(angle_rads)], axis=-1)
    return np.asarray(pe, dtype=np.float32)


_PE = _positional_encoding(MAX_SEQ_LEN, D_MODEL)


def _sc_body(x_hbm, table_hbm, pe_hbm, out_hbm,
             idx_all, rows0, rows1, ob0, ob1, pe_v,
             sg0, sg1, so0, so1):
    rows = (rows0, rows1)
    ob = (ob0, ob1)
    sg = (sg0, sg1)
    so = (so0, so1)

    c = lax.axis_index("c")
    s = lax.axis_index("s")
    wid = s * NUM_CORES + c
    row0 = wid * CHUNKS

    pltpu.sync_copy(pe_hbm, pe_v)
    pltpu.sync_copy(x_hbm.at[wid], idx_all)
    pltpu.async_copy(table_hbm.at[idx_all.at[0]], rows0, sg0)

    def chunk_step(g, b):
        @pl.when(g + 1 < CHUNKS)
        def _():
            pltpu.async_copy(table_hbm.at[idx_all.at[g + 1]], rows[1 - b],
                             sg[1 - b])

        pltpu.make_async_copy(table_hbm.at[idx_all.at[g]], rows[b],
                              sg[b]).wait()

        @pl.when(g >= 2)
        def _():
            pltpu.make_async_copy(ob[b], out_hbm.at[pl.ds(0, CHUNK)],
                                  so[b]).wait()

        def tok(t, _):
            pos = lax.rem(g * CHUNK + t, MAX_SEQ_LEN)
            for dv in range(DVECS):
                sl = pl.ds(dv * LANES, LANES)
                ob[b][t, sl] = rows[b][t, sl] * SCALE + pe_v[pos, sl]
            return 0

        lax.fori_loop(0, CHUNK, tok, 0)

        pltpu.async_copy(ob[b], out_hbm.at[pl.ds((row0 + g) * CHUNK, CHUNK)],
                         so[b])

    def outer(i, _):
        chunk_step(2 * i, 0)
        chunk_step(2 * i + 1, 1)
        return 0

    lax.fori_loop(0, CHUNKS // 2, outer, 0)

    pltpu.make_async_copy(ob0, out_hbm.at[pl.ds(0, CHUNK)], so0).wait()
    pltpu.make_async_copy(ob1, out_hbm.at[pl.ds(0, CHUNK)], so1).wait()


@jax.jit
def kernel(x, table):
    x3d = x.reshape(NUM_WORKERS, CHUNKS, CHUNK)
    mesh = plsc.VectorSubcoreMesh(core_axis_name="c", subcore_axis_name="s")
    run = pl.kernel(
        _sc_body,
        out_type=jax.ShapeDtypeStruct((TOKENS, D_MODEL), jnp.float32),
        mesh=mesh,
        scratch_types=[
            pltpu.VMEM((CHUNKS, CHUNK), jnp.int32),
            pltpu.VMEM((CHUNK, D_MODEL), jnp.float32),
            pltpu.VMEM((CHUNK, D_MODEL), jnp.float32),
            pltpu.VMEM((CHUNK, D_MODEL), jnp.float32),
            pltpu.VMEM((CHUNK, D_MODEL), jnp.float32),
            pltpu.VMEM((MAX_SEQ_LEN, D_MODEL), jnp.float32),
            pltpu.SemaphoreType.DMA,
            pltpu.SemaphoreType.DMA,
            pltpu.SemaphoreType.DMA,
            pltpu.SemaphoreType.DMA,
        ],
    )
    out = run(x3d, table, jnp.asarray(_PE))
    return out.reshape(BATCH, SEQ_LEN, D_MODEL)
